# baseline (speedup 1.0000x reference)
import functools

import jax
import jax.numpy as jnp
from jax import lax
from jax.experimental import pallas as pl
from jax.experimental.pallas import tpu as pltpu
from jax.experimental.pallas import tpu_sc as plsc

B, C, G, S, L = 32, 16, 2048, 8, 3
NC, NS = 2, 16
NW = NC * NS
BW = B // 2
PW = BW // 2
SL = S * L
NT = G // 16


def _sc_body(xpk_hbm, i_hbm, out_hbm, x_v, idx_v, out_v, sem_x, sem_i):
    w = lax.axis_index("s") * NC + lax.axis_index("c")
    c = w // 2
    b0 = (w - c * 2) * BW
    p0 = b0 // 2
    cp_x = pltpu.async_copy(xpk_hbm.at[pl.ds(p0 * G, PW * G)], x_v, sem_x)
    cp_i = pltpu.async_copy(i_hbm.at[c], idx_v, sem_i)
    cp_x.wait()
    cp_i.wait()

    @plsc.parallel_loop(0, NT, step=1, unroll=2)
    def t_body(t):
        og = t * 16
        iv = [[idx_v[pl.ds((s * L + l) * G + og, 16)]
               for l in range(L)] for s in range(S)]

        @plsc.parallel_loop(0, PW, step=1, unroll=8)
        def p_body(p):
            poff = p * G
            acc_e = acc_o = None
            for s in range(S):
                pr = (
                    plsc.bitcast(plsc.load_gather(x_v, [iv[s][0] + poff]),
                                 jnp.bfloat16)
                    * plsc.bitcast(plsc.load_gather(x_v, [iv[s][1] + poff]),
                                   jnp.bfloat16)
                    * plsc.bitcast(plsc.load_gather(x_v, [iv[s][2] + poff]),
                                   jnp.bfloat16)
                )
                e, o = plsc.unpack(pr, format=plsc.PackFormat.INTERLEAVED)
                acc_e = e if acc_e is None else acc_e + e
                acc_o = o if acc_o is None else acc_o + o
            out_v[2 * p, pl.ds(og, 16)] = acc_e
            out_v[2 * p + 1, pl.ds(og, 16)] = acc_o

    pltpu.sync_copy(out_v, out_hbm.at[pl.ds(c * B + b0, BW)])


_sc_call = functools.partial(
    pl.kernel,
    out_type=jax.ShapeDtypeStruct((C * B, G), jnp.float32),
    mesh=plsc.VectorSubcoreMesh(core_axis_name="c", subcore_axis_name="s"),
    compiler_params=pltpu.CompilerParams(needs_layout_passes=False),
    scratch_types=[
        pltpu.VMEM((PW * G,), jnp.int32),
        pltpu.VMEM((G * SL,), jnp.int32),
        pltpu.VMEM((BW, G), jnp.float32),
        pltpu.SemaphoreType.DMA,
        pltpu.SemaphoreType.DMA,
    ],
)(_sc_body)


def kernel(x, I):
    xb = jax.lax.bitcast_convert_type(x.astype(jnp.bfloat16), jnp.uint16)
    xpk = jax.lax.bitcast_convert_type(
        xb[0::2].astype(jnp.uint32) | (xb[1::2].astype(jnp.uint32) << 16),
        jnp.int32)
    I_p = I.transpose(0, 2, 3, 1).reshape(C, SL * G)
    out = _sc_call(xpk.reshape(PW * 2 * G), I_p)
    return out.reshape(C, B, G)


# speedup vs baseline: 1.0845x; 1.0845x over previous
import functools

import jax
import jax.numpy as jnp
from jax import lax
from jax.experimental import pallas as pl
from jax.experimental.pallas import tpu as pltpu
from jax.experimental.pallas import tpu_sc as plsc

B, C, G, S, L = 32, 16, 2048, 8, 3
NC, NS = 2, 16
NW = NC * NS
BW = B // 2
PW = BW // 2
SL = S * L
NT = G // 16


def _sc_body(xpk_hbm, i_hbm, out_hbm, x_v, idx_v, out_v, sem_x, sem_i):
    w = lax.axis_index("s") * NC + lax.axis_index("c")
    c = w // 2
    b0 = (w - c * 2) * BW
    p0 = b0 // 2
    cp_x = pltpu.async_copy(xpk_hbm.at[pl.ds(p0 * G, PW * G)], x_v, sem_x)
    cp_i = pltpu.async_copy(i_hbm.at[c], idx_v, sem_i)
    cp_x.wait()
    cp_i.wait()

    def t_body(t, _):
        og = t * 16
        iv = [[idx_v[pl.ds((s * L + l) * G + og, 16)]
               for l in range(L)] for s in range(S)]

        @plsc.parallel_loop(0, PW, step=1, unroll=8)
        def p_body(p):
            poff = p * G
            acc_e = acc_o = None
            for s in range(S):
                pr = (
                    plsc.bitcast(plsc.load_gather(x_v, [iv[s][0] + poff]),
                                 jnp.bfloat16)
                    * plsc.bitcast(plsc.load_gather(x_v, [iv[s][1] + poff]),
                                   jnp.bfloat16)
                    * plsc.bitcast(plsc.load_gather(x_v, [iv[s][2] + poff]),
                                   jnp.bfloat16)
                )
                e, o = plsc.unpack(pr, format=plsc.PackFormat.INTERLEAVED)
                acc_e = e if acc_e is None else acc_e + e
                acc_o = o if acc_o is None else acc_o + o
            out_v[2 * p, pl.ds(og, 16)] = acc_e
            out_v[2 * p + 1, pl.ds(og, 16)] = acc_o

        return 0

    lax.fori_loop(0, NT, t_body, 0)
    pltpu.sync_copy(out_v, out_hbm.at[pl.ds(c * B + b0, BW)])


_sc_call = functools.partial(
    pl.kernel,
    out_type=jax.ShapeDtypeStruct((C * B, G), jnp.float32),
    mesh=plsc.VectorSubcoreMesh(core_axis_name="c", subcore_axis_name="s"),
    compiler_params=pltpu.CompilerParams(needs_layout_passes=False),
    scratch_types=[
        pltpu.VMEM((PW * G,), jnp.int32),
        pltpu.VMEM((G * SL,), jnp.int32),
        pltpu.VMEM((BW, G), jnp.float32),
        pltpu.SemaphoreType.DMA,
        pltpu.SemaphoreType.DMA,
    ],
)(_sc_body)


def kernel(x, I):
    xb = jax.lax.bitcast_convert_type(x.astype(jnp.bfloat16), jnp.uint16)
    xpk = jax.lax.bitcast_convert_type(
        xb[0::2].astype(jnp.uint32) | (xb[1::2].astype(jnp.uint32) << 16),
        jnp.int32)
    I_p = I.transpose(0, 2, 3, 1).reshape(C, SL * G)
    out = _sc_call(xpk.reshape(PW * 2 * G), I_p)
    return out.reshape(C, B, G)
